# baseline (speedup 1.0000x reference)
import functools

import jax
import jax.numpy as jnp
from jax import lax
from jax.experimental import pallas as pl
from jax.experimental.pallas import tpu as pltpu
from jax.experimental.pallas import tpu_sc as plsc

_NC, _NS, _L = 2, 16, 16
_SLAB = 128


def _down_body(x_ref, wd1, bd1, wd2, bd2, wd3, bd3, pin, pinb, cb, dT_ref):
    f32 = jnp.float32
    xb = x_ref[...]
    h = jnp.maximum(jnp.dot(xb, wd1[...], preferred_element_type=f32) + bd1[...], 0.0)
    h = jnp.maximum(jnp.dot(h, wd2[...], preferred_element_type=f32) + bd2[...], 0.0)
    h = jnp.dot(h, wd3[...], preferred_element_type=f32) + bd3[...]
    E, K, CD = cb.shape
    bB = h.shape[0]
    nslab = bB // _SLAB
    rows = []
    for i in range(E):
        z = jnp.dot(h, pin[i], preferred_element_type=f32) + pinb[i]
        cbi = cb[i]
        cb2 = jnp.sum(cbi * cbi, axis=1)[:, None]
        slabs = []
        for t in range(nslab):
            zt = z[t * _SLAB:(t + 1) * _SLAB]
            cross = lax.dot_general(cbi, zt, (((1,), (1,)), ((), ())),
                                    preferred_element_type=f32)
            slabs.append(cb2 - 2.0 * cross)
        rows.append(jnp.stack(slabs, axis=0))
    dT_ref[...] = jnp.stack(rows, axis=0)


def _decode_body(cb_ref, pout, poutb, m_ref):
    f32 = jnp.float32
    E, K, CD = cb_ref.shape
    scale = 1.0 / E
    for i in range(E):
        m_ref[i] = (jnp.dot(cb_ref[i], pout[i], preferred_element_type=f32)
                    + poutb[i]) * scale


def _up_body(q_ref, wu1, bu1, wu2, bu2, wu3, bu3, u_ref):
    f32 = jnp.float32
    E = q_ref.shape[0]
    hq = q_ref[0]
    for i in range(1, E):
        hq = hq + q_ref[i]
    u = jnp.maximum(jnp.dot(hq, wu1[...], preferred_element_type=f32) + bu1[...], 0.0)
    u = jnp.maximum(jnp.dot(u, wu2[...], preferred_element_type=f32) + bu2[...], 0.0)
    u = jnp.dot(u.astype(jnp.bfloat16), wu3[...],
                preferred_element_type=f32) + bu3[...]
    u_ref[...] = jnp.clip(u, -1.0, 1.0)


def _sc_argmin_gather(dT, m_flat, E, K, H):
    B = dT.shape[1] * _SLAB
    NW = _NC * _NS
    per_w = (B // _SLAB) // NW
    nsub = _SLAB // _L
    mesh = plsc.VectorSubcoreMesh(core_axis_name="c", subcore_axis_name="s")

    n_items = E * per_w

    @functools.partial(
        pl.kernel, mesh=mesh,
        out_type=[jax.ShapeDtypeStruct((E, B), jnp.int32),
                  jax.ShapeDtypeStruct((E, B, H), jnp.float32)],
        scratch_types=[pltpu.VMEM((K, _SLAB), jnp.float32),
                       pltpu.VMEM((K, _SLAB), jnp.float32),
                       pltpu.VMEM((1, _SLAB), jnp.int32),
                       pltpu.VMEM((1, _SLAB), jnp.int32),
                       pltpu.VMEM((_SLAB,), jnp.int32),
                       pltpu.VMEM((_SLAB,), jnp.int32),
                       pltpu.VMEM((_SLAB, H), jnp.float32),
                       pltpu.VMEM((_SLAB, H), jnp.float32),
                       pltpu.VMEM_SHARED((1024, H), jnp.float32),
                       pltpu.SemaphoreType.DMA,
                       pltpu.SemaphoreType.DMA,
                       pltpu.SemaphoreType.DMA,
                       pltpu.SemaphoreType.DMA,
                       pltpu.SemaphoreType.DMA,
                       pltpu.SemaphoreType.DMA,
                       pltpu.SemaphoreType.DMA,
                       pltpu.SemaphoreType.DMA],
    )
    def k(dT_hbm, m_hbm, idx_hbm, q_hbm,
          d_v0, d_v1, i_v0, i_v1, g_v0, g_v1, r_v0, r_v1, m_sh,
          s_in0, s_in1, s_ix0, s_ix1, s_g0, s_g1, s_q0, s_q1):
        wid = lax.axis_index("s") * _NC + lax.axis_index("c")
        @pl.when(lax.axis_index("s") == 0)
        def _():
            pltpu.sync_copy(m_hbm, m_sh.at[pl.ds(0, E * K)])
        plsc.subcore_barrier()
        d_v = (d_v0, d_v1)
        i_v = (i_v0, i_v1)
        g_v = (g_v0, g_v1)
        r_v = (r_v0, r_v1)
        s_in = (s_in0, s_in1)
        s_ix = (s_ix0, s_ix1)
        s_g = (s_g0, s_g1)
        s_q = (s_q0, s_q1)

        def slab_src(item):
            e, j = divmod(item, per_w)
            base = (wid * per_w + j) * _SLAB
            return e, base

        def in_copy(item, p):
            e, base = slab_src(item)
            return pltpu.make_async_copy(
                dT_hbm.at[e, base // _SLAB], d_v[p], s_in[p])

        def ix_copy(item, p):
            e, base = slab_src(item)
            return pltpu.make_async_copy(
                i_v[p], idx_hbm.at[pl.ds(e, 1), pl.ds(base, _SLAB)], s_ix[p])

        def q_copy(item, p):
            e, base = slab_src(item)
            return pltpu.make_async_copy(
                r_v[p], q_hbm.at[e, pl.ds(base, _SLAB), :], s_q[p])

        UN = 4
        inf = jnp.full((_L,), jnp.inf, jnp.float32)
        zero = jnp.zeros((_L,), jnp.int32)

        in_copy(0, 0).start()
        for item in range(n_items):
            p = item % 2
            e, base = slab_src(item)
            if item + 1 < n_items:
                in_copy(item + 1, 1 - p).start()
            in_copy(item, p).wait()

            def body(kk, carry, _d=d_v[p]):
                outs = []
                for s in range(nsub):
                    best, besti = carry[s]
                    for u in range(UN):
                        krow = kk * UN + u
                        val = _d[krow, pl.ds(s * _L, _L)]
                        pred = val < best
                        best = jnp.where(pred, val, best)
                        besti = jnp.where(
                            pred, jnp.full((_L,), krow, jnp.int32), besti)
                    outs.append((best, besti))
                return tuple(outs)

            carry = lax.fori_loop(
                0, K // UN, body, tuple((inf, zero) for _ in range(nsub)))

            if item >= 2:
                ix_copy(item - 2, p).wait()
            for s in range(nsub):
                sl = pl.ds(s * _L, _L)
                i_v[p][0, sl] = carry[s][1]
                g_v[p][sl] = carry[s][1] + e * K
            ix_copy(item, p).start()

            if item >= 2:
                q_copy(item - 2, p).wait()
            pltpu.make_async_copy(m_sh.at[g_v[p]], r_v[p], s_g[p]).start()
            if item >= 1:
                pm = (item - 1) % 2
                pltpu.make_async_copy(
                    m_sh.at[g_v[pm]], r_v[pm], s_g[pm]).wait()
                q_copy(item - 1, pm).start()

        pl_last = (n_items - 1) % 2
        pltpu.make_async_copy(
            m_sh.at[g_v[pl_last]], r_v[pl_last], s_g[pl_last]).wait()
        q_copy(n_items - 1, pl_last).start()
        for item in (n_items - 2, n_items - 1):
            p = item % 2
            ix_copy(item, p).wait()
            q_copy(item, p).wait()

    return k(dT, m_flat)


def kernel(x, Wd1, bd1, Wd2, bd2, Wd3, bd3, Pin, Pin_b, codebooks, Pout,
           Pout_b, Wu1, bu1, Wu2, bu2, Wu3, bu3):
    B, D = x.shape
    H = Wd3.shape[1]
    E, K, CD = codebooks.shape
    bB = 256
    NB = 2

    down_w = (Wd1, bd1, Wd2, bd2, Wd3, bd3, Pin, Pin_b, codebooks)
    up_w = (Wu1, bu1, Wu2, bu2, Wu3.astype(jnp.bfloat16), bu3)

    def full(a):
        return pl.BlockSpec(a.shape, lambda i: (0,) * a.ndim)

    M = pl.pallas_call(
        _decode_body,
        in_specs=[pl.BlockSpec(a.shape, lambda *_, _n=a.ndim: (0,) * _n)
                  for a in (codebooks, Pout, Pout_b)],
        out_specs=pl.BlockSpec((E, K, H), lambda *_: (0, 0, 0)),
        out_shape=jax.ShapeDtypeStruct((E, K, H), jnp.float32),
    )(codebooks, Pout, Pout_b)
    m_flat = M.reshape(E * K, H)

    Bc = B // NB
    u_parts, idx_parts = [], []
    for nb in range(NB):
        xc = lax.slice_in_dim(x, nb * Bc, (nb + 1) * Bc, axis=0)
        dT = pl.pallas_call(
            _down_body,
            grid=(Bc // bB,),
            in_specs=[pl.BlockSpec((bB, D), lambda i: (i, 0))] +
                     [full(w) for w in down_w],
            out_specs=pl.BlockSpec((E, bB // _SLAB, K, _SLAB),
                                   lambda i: (0, i, 0, 0)),
            out_shape=jax.ShapeDtypeStruct((E, Bc // _SLAB, K, _SLAB),
                                           jnp.float32),
            compiler_params=pltpu.CompilerParams(
                dimension_semantics=("arbitrary",)),
        )(xc, *down_w)

        idx_c, q_c = _sc_argmin_gather(dT, m_flat, E, K, H)

        u_c = pl.pallas_call(
            _up_body,
            grid=(Bc // bB,),
            in_specs=[pl.BlockSpec((E, bB, H), lambda i: (0, i, 0))] +
                     [full(w) for w in up_w],
            out_specs=pl.BlockSpec((bB, D), lambda i: (i, 0)),
            out_shape=jax.ShapeDtypeStruct((Bc, D), jnp.float32),
            compiler_params=pltpu.CompilerParams(
                dimension_semantics=("arbitrary",)),
        )(q_c, *up_w)
        u_parts.append(u_c)
        idx_parts.append(idx_c)

    u = u_parts[0] if NB == 1 else jnp.concatenate(u_parts, axis=0)
    idx = idx_parts[0] if NB == 1 else jnp.concatenate(idx_parts, axis=1)
    return u, idx.T, jnp.zeros((), jnp.float32)


# speedup vs baseline: 2.2758x; 2.2758x over previous
import functools

import jax
import jax.numpy as jnp
from jax import lax
from jax.experimental import pallas as pl
from jax.experimental.pallas import tpu as pltpu
from jax.experimental.pallas import tpu_sc as plsc

_NC, _NS, _L = 2, 16, 16
_SLAB = 128


def _down_body(x_ref, wd1, bd1, wd2, bd2, wd3, bd3, pin, pinb, cb, dT_ref):
    f32 = jnp.float32
    xb = x_ref[...]
    h = jnp.maximum(jnp.dot(xb, wd1[...], preferred_element_type=f32) + bd1[...], 0.0)
    h = jnp.maximum(jnp.dot(h, wd2[...], preferred_element_type=f32) + bd2[...], 0.0)
    h = jnp.dot(h, wd3[...], preferred_element_type=f32) + bd3[...]
    E, K, CD = cb.shape
    bB = h.shape[0]
    nslab = bB // _SLAB
    rows = []
    for i in range(E):
        z = jnp.dot(h, pin[i], preferred_element_type=f32) + pinb[i]
        cbi = cb[i]
        cb2 = jnp.sum(cbi * cbi, axis=1)[:, None]
        slabs = []
        for t in range(nslab):
            zt = z[t * _SLAB:(t + 1) * _SLAB]
            cross = lax.dot_general(cbi, zt, (((1,), (1,)), ((), ())),
                                    preferred_element_type=f32)
            slabs.append(cb2 - 2.0 * cross)
        rows.append(jnp.stack(slabs, axis=0))
    dT_ref[...] = jnp.stack(rows, axis=0)


def _decode_body(cb_ref, pout, poutb, m_ref):
    f32 = jnp.float32
    E, K, CD = cb_ref.shape
    scale = 1.0 / E
    for i in range(E):
        m_ref[i] = (jnp.dot(cb_ref[i], pout[i], preferred_element_type=f32)
                    + poutb[i]) * scale


def _up_body(q_ref, wu1, bu1, wu2, bu2, wu3, bu3, u_ref):
    f32 = jnp.float32
    E = q_ref.shape[0]
    hq = q_ref[0]
    for i in range(1, E):
        hq = hq + q_ref[i]
    u = jnp.maximum(jnp.dot(hq, wu1[...], preferred_element_type=f32) + bu1[...], 0.0)
    u = jnp.maximum(jnp.dot(u, wu2[...], preferred_element_type=f32) + bu2[...], 0.0)
    u = jnp.dot(u, wu3[...], preferred_element_type=f32) + bu3[...]
    u_ref[...] = jnp.clip(u, -1.0, 1.0)


def _sc_argmin_gather(dT, m_flat, E, K, H):
    B = dT.shape[1] * _SLAB
    NW = _NC * _NS
    per_w = (B // _SLAB) // NW
    nsub = _SLAB // _L
    mesh = plsc.VectorSubcoreMesh(core_axis_name="c", subcore_axis_name="s")

    n_items = E * per_w

    @functools.partial(
        pl.kernel, mesh=mesh,
        out_type=[jax.ShapeDtypeStruct((E, B), jnp.int32),
                  jax.ShapeDtypeStruct((E, B, H), jnp.float32)],
        scratch_types=[pltpu.VMEM((K, _SLAB), jnp.float32),
                       pltpu.VMEM((K, _SLAB), jnp.float32),
                       pltpu.VMEM((1, _SLAB), jnp.int32),
                       pltpu.VMEM((1, _SLAB), jnp.int32),
                       pltpu.VMEM((_SLAB,), jnp.int32),
                       pltpu.VMEM((_SLAB,), jnp.int32),
                       pltpu.VMEM((_SLAB, H), jnp.float32),
                       pltpu.VMEM((_SLAB, H), jnp.float32),
                       pltpu.VMEM_SHARED((1024, H), jnp.float32),
                       pltpu.SemaphoreType.DMA,
                       pltpu.SemaphoreType.DMA,
                       pltpu.SemaphoreType.DMA,
                       pltpu.SemaphoreType.DMA,
                       pltpu.SemaphoreType.DMA,
                       pltpu.SemaphoreType.DMA,
                       pltpu.SemaphoreType.DMA,
                       pltpu.SemaphoreType.DMA],
    )
    def k(dT_hbm, m_hbm, idx_hbm, q_hbm,
          d_v0, d_v1, i_v0, i_v1, g_v0, g_v1, r_v0, r_v1, m_sh,
          s_in0, s_in1, s_ix0, s_ix1, s_g0, s_g1, s_q0, s_q1):
        wid = lax.axis_index("s") * _NC + lax.axis_index("c")
        @pl.when(lax.axis_index("s") == 0)
        def _():
            pltpu.sync_copy(m_hbm, m_sh.at[pl.ds(0, E * K)])
        plsc.subcore_barrier()
        d_v = (d_v0, d_v1)
        i_v = (i_v0, i_v1)
        g_v = (g_v0, g_v1)
        r_v = (r_v0, r_v1)
        s_in = (s_in0, s_in1)
        s_ix = (s_ix0, s_ix1)
        s_g = (s_g0, s_g1)
        s_q = (s_q0, s_q1)

        def slab_src(item):
            e, j = divmod(item, per_w)
            base = (wid * per_w + j) * _SLAB
            return e, base

        def in_copy(item, p):
            e, base = slab_src(item)
            return pltpu.make_async_copy(
                dT_hbm.at[e, base // _SLAB], d_v[p], s_in[p])

        def ix_copy(item, p):
            e, base = slab_src(item)
            return pltpu.make_async_copy(
                i_v[p], idx_hbm.at[pl.ds(e, 1), pl.ds(base, _SLAB)], s_ix[p])

        def q_copy(item, p):
            e, base = slab_src(item)
            return pltpu.make_async_copy(
                r_v[p], q_hbm.at[e, pl.ds(base, _SLAB), :], s_q[p])

        UN = 4
        inf = jnp.full((_L,), jnp.inf, jnp.float32)
        zero = jnp.zeros((_L,), jnp.int32)

        in_copy(0, 0).start()
        for item in range(n_items):
            p = item % 2
            e, base = slab_src(item)
            if item + 1 < n_items:
                in_copy(item + 1, 1 - p).start()
            in_copy(item, p).wait()

            def body(kk, carry, _d=d_v[p]):
                outs = []
                for s in range(nsub):
                    best, besti = carry[s]
                    for u in range(UN):
                        krow = kk * UN + u
                        val = _d[krow, pl.ds(s * _L, _L)]
                        pred = val < best
                        best = jnp.where(pred, val, best)
                        besti = jnp.where(
                            pred, jnp.full((_L,), krow, jnp.int32), besti)
                    outs.append((best, besti))
                return tuple(outs)

            carry = lax.fori_loop(
                0, K // UN, body, tuple((inf, zero) for _ in range(nsub)))

            if item >= 2:
                ix_copy(item - 2, p).wait()
            for s in range(nsub):
                sl = pl.ds(s * _L, _L)
                i_v[p][0, sl] = carry[s][1]
                g_v[p][sl] = carry[s][1] + e * K
            ix_copy(item, p).start()

            if item >= 2:
                q_copy(item - 2, p).wait()
            pltpu.make_async_copy(m_sh.at[g_v[p]], r_v[p], s_g[p]).start()
            if item >= 1:
                pm = (item - 1) % 2
                pltpu.make_async_copy(
                    m_sh.at[g_v[pm]], r_v[pm], s_g[pm]).wait()
                q_copy(item - 1, pm).start()

        pl_last = (n_items - 1) % 2
        pltpu.make_async_copy(
            m_sh.at[g_v[pl_last]], r_v[pl_last], s_g[pl_last]).wait()
        q_copy(n_items - 1, pl_last).start()
        for item in (n_items - 2, n_items - 1):
            p = item % 2
            ix_copy(item, p).wait()
            q_copy(item, p).wait()

    return k(dT, m_flat)


def kernel(x, Wd1, bd1, Wd2, bd2, Wd3, bd3, Pin, Pin_b, codebooks, Pout,
           Pout_b, Wu1, bu1, Wu2, bu2, Wu3, bu3):
    B, D = x.shape
    H = Wd3.shape[1]
    E, K, CD = codebooks.shape
    bB = 512
    NB = 1

    down_w = (Wd1, bd1, Wd2, bd2, Wd3, bd3, Pin, Pin_b, codebooks)
    up_w = (Wu1, bu1, Wu2, bu2, Wu3, bu3)

    def full(a):
        return pl.BlockSpec(a.shape, lambda i: (0,) * a.ndim)

    M = pl.pallas_call(
        _decode_body,
        in_specs=[pl.BlockSpec(a.shape, lambda *_, _n=a.ndim: (0,) * _n)
                  for a in (codebooks, Pout, Pout_b)],
        out_specs=pl.BlockSpec((E, K, H), lambda *_: (0, 0, 0)),
        out_shape=jax.ShapeDtypeStruct((E, K, H), jnp.float32),
    )(codebooks, Pout, Pout_b)
    m_flat = M.reshape(E * K, H)

    Bc = B // NB
    u_parts, idx_parts = [], []
    for nb in range(NB):
        xc = lax.slice_in_dim(x, nb * Bc, (nb + 1) * Bc, axis=0)
        dT = pl.pallas_call(
            _down_body,
            grid=(Bc // bB,),
            in_specs=[pl.BlockSpec((bB, D), lambda i: (i, 0))] +
                     [full(w) for w in down_w],
            out_specs=pl.BlockSpec((E, bB // _SLAB, K, _SLAB),
                                   lambda i: (0, i, 0, 0)),
            out_shape=jax.ShapeDtypeStruct((E, Bc // _SLAB, K, _SLAB),
                                           jnp.float32),
            compiler_params=pltpu.CompilerParams(
                dimension_semantics=("arbitrary",)),
        )(xc, *down_w)

        idx_c, q_c = _sc_argmin_gather(dT, m_flat, E, K, H)

        u_c = pl.pallas_call(
            _up_body,
            grid=(Bc // bB,),
            in_specs=[pl.BlockSpec((E, bB, H), lambda i: (0, i, 0))] +
                     [full(w) for w in up_w],
            out_specs=pl.BlockSpec((bB, D), lambda i: (i, 0)),
            out_shape=jax.ShapeDtypeStruct((Bc, D), jnp.float32),
            compiler_params=pltpu.CompilerParams(
                dimension_semantics=("arbitrary",)),
        )(q_c, *up_w)
        u_parts.append(u_c)
        idx_parts.append(idx_c)

    u = u_parts[0] if NB == 1 else jnp.concatenate(u_parts, axis=0)
    idx = idx_parts[0] if NB == 1 else jnp.concatenate(idx_parts, axis=1)
    return u, idx.T, jnp.zeros((), jnp.float32)


# speedup vs baseline: 2.4316x; 1.0685x over previous
import functools

import jax
import jax.numpy as jnp
from jax import lax
from jax.experimental import pallas as pl
from jax.experimental.pallas import tpu as pltpu
from jax.experimental.pallas import tpu_sc as plsc

_NC, _NS, _L = 2, 16, 16
_SLAB = 128


def _down_body(x_ref, wd1, bd1, wd2, bd2, wd3, bd3, pin, pinb, cb, dT_ref):
    f32 = jnp.float32
    xb = x_ref[...]
    h = jnp.maximum(jnp.dot(xb, wd1[...], preferred_element_type=f32) + bd1[...], 0.0)
    h = jnp.maximum(jnp.dot(h, wd2[...], preferred_element_type=f32) + bd2[...], 0.0)
    h = jnp.dot(h, wd3[...], preferred_element_type=f32) + bd3[...]
    E, K, CD = cb.shape
    bB = h.shape[0]
    nslab = bB // _SLAB
    rows = []
    for i in range(E):
        z = jnp.dot(h, pin[i], preferred_element_type=f32) + pinb[i]
        cbi = cb[i]
        cb2 = jnp.sum(cbi * cbi, axis=1)[:, None]
        slabs = []
        for t in range(nslab):
            zt = z[t * _SLAB:(t + 1) * _SLAB]
            cross = lax.dot_general(cbi, zt, (((1,), (1,)), ((), ())),
                                    preferred_element_type=f32)
            slabs.append(cb2 - 2.0 * cross)
        rows.append(jnp.stack(slabs, axis=0))
    dT_ref[...] = jnp.stack(rows, axis=0)


def _decode_body(cb_ref, pout, poutb, m_ref):
    f32 = jnp.float32
    E, K, CD = cb_ref.shape
    scale = 1.0 / E
    for i in range(E):
        m_ref[i] = (jnp.dot(cb_ref[i], pout[i], preferred_element_type=f32)
                    + poutb[i]) * scale


def _up_body(q_ref, wu1, bu1, wu2, bu2, wu3, bu3, u_ref):
    f32 = jnp.float32
    E = q_ref.shape[0]
    hq = q_ref[0]
    for i in range(1, E):
        hq = hq + q_ref[i]
    u = jnp.maximum(jnp.dot(hq, wu1[...], preferred_element_type=f32) + bu1[...], 0.0)
    u = jnp.maximum(jnp.dot(u, wu2[...], preferred_element_type=f32) + bu2[...], 0.0)
    u = jnp.dot(u, wu3[...], preferred_element_type=f32) + bu3[...]
    u_ref[...] = jnp.clip(u, -1.0, 1.0)


def _sc_argmin_gather(dT, m_flat, E, K, H):
    B = dT.shape[1] * _SLAB
    NW = _NC * _NS
    per_w = (B // _SLAB) // NW
    nsub = _SLAB // _L
    mesh = plsc.VectorSubcoreMesh(core_axis_name="c", subcore_axis_name="s")

    n_items = E * per_w

    @functools.partial(
        pl.kernel, mesh=mesh,
        out_type=[jax.ShapeDtypeStruct((E, B), jnp.int32),
                  jax.ShapeDtypeStruct((E, B, H), jnp.float32)],
        scratch_types=[pltpu.VMEM((K, _SLAB), jnp.float32),
                       pltpu.VMEM((K, _SLAB), jnp.float32),
                       pltpu.VMEM((1, _SLAB), jnp.int32),
                       pltpu.VMEM((1, _SLAB), jnp.int32),
                       pltpu.VMEM((_SLAB,), jnp.int32),
                       pltpu.VMEM((_SLAB,), jnp.int32),
                       pltpu.VMEM((_SLAB, H), jnp.float32),
                       pltpu.VMEM((_SLAB, H), jnp.float32),
                       pltpu.VMEM_SHARED((1024, H), jnp.float32),
                       pltpu.SemaphoreType.DMA,
                       pltpu.SemaphoreType.DMA,
                       pltpu.SemaphoreType.DMA,
                       pltpu.SemaphoreType.DMA,
                       pltpu.SemaphoreType.DMA,
                       pltpu.SemaphoreType.DMA,
                       pltpu.SemaphoreType.DMA,
                       pltpu.SemaphoreType.DMA],
    )
    def k(dT_hbm, m_hbm, idx_hbm, q_hbm,
          d_v0, d_v1, i_v0, i_v1, g_v0, g_v1, r_v0, r_v1, m_sh,
          s_in0, s_in1, s_ix0, s_ix1, s_g0, s_g1, s_q0, s_q1):
        wid = lax.axis_index("s") * _NC + lax.axis_index("c")
        @pl.when(lax.axis_index("s") == 0)
        def _():
            pltpu.sync_copy(m_hbm, m_sh.at[pl.ds(0, E * K)])
        plsc.subcore_barrier()
        d_v = (d_v0, d_v1)
        i_v = (i_v0, i_v1)
        g_v = (g_v0, g_v1)
        r_v = (r_v0, r_v1)
        s_in = (s_in0, s_in1)
        s_ix = (s_ix0, s_ix1)
        s_g = (s_g0, s_g1)
        s_q = (s_q0, s_q1)

        def slab_src(item):
            e, j = divmod(item, per_w)
            base = (wid * per_w + j) * _SLAB
            return e, base

        def in_copy(item, p):
            e, base = slab_src(item)
            return pltpu.make_async_copy(
                dT_hbm.at[e, base // _SLAB], d_v[p], s_in[p])

        def ix_copy(item, p):
            e, base = slab_src(item)
            return pltpu.make_async_copy(
                i_v[p], idx_hbm.at[pl.ds(e, 1), pl.ds(base, _SLAB)], s_ix[p])

        def q_copy(item, p):
            e, base = slab_src(item)
            return pltpu.make_async_copy(
                r_v[p], q_hbm.at[e, pl.ds(base, _SLAB), :], s_q[p])

        UN = 4
        inf = jnp.full((_L,), jnp.inf, jnp.float32)
        zero = jnp.zeros((_L,), jnp.int32)

        in_copy(0, 0).start()
        for item in range(n_items):
            p = item % 2
            e, base = slab_src(item)
            if item + 1 < n_items:
                in_copy(item + 1, 1 - p).start()
            in_copy(item, p).wait()

            def body(kk, carry, _d=d_v[p]):
                outs = []
                for s in range(nsub):
                    best, besti = carry[s]
                    for u in range(UN):
                        krow = kk * UN + u
                        val = _d[krow, pl.ds(s * _L, _L)]
                        pred = val < best
                        best = jnp.where(pred, val, best)
                        besti = jnp.where(
                            pred, jnp.full((_L,), krow, jnp.int32), besti)
                    outs.append((best, besti))
                return tuple(outs)

            carry = lax.fori_loop(
                0, K // UN, body, tuple((inf, zero) for _ in range(nsub)))

            if item >= 2:
                ix_copy(item - 2, p).wait()
            for s in range(nsub):
                sl = pl.ds(s * _L, _L)
                i_v[p][0, sl] = carry[s][1]
                g_v[p][sl] = carry[s][1] + e * K
            ix_copy(item, p).start()

            if item >= 2:
                q_copy(item - 2, p).wait()
            pltpu.make_async_copy(m_sh.at[g_v[p]], r_v[p], s_g[p]).start()
            if item >= 1:
                pm = (item - 1) % 2
                pltpu.make_async_copy(
                    m_sh.at[g_v[pm]], r_v[pm], s_g[pm]).wait()
                q_copy(item - 1, pm).start()

        pl_last = (n_items - 1) % 2
        pltpu.make_async_copy(
            m_sh.at[g_v[pl_last]], r_v[pl_last], s_g[pl_last]).wait()
        q_copy(n_items - 1, pl_last).start()
        for item in (n_items - 2, n_items - 1):
            p = item % 2
            ix_copy(item, p).wait()
            q_copy(item, p).wait()

    return k(dT, m_flat)


def kernel(x, Wd1, bd1, Wd2, bd2, Wd3, bd3, Pin, Pin_b, codebooks, Pout,
           Pout_b, Wu1, bu1, Wu2, bu2, Wu3, bu3):
    B, D = x.shape
    H = Wd3.shape[1]
    E, K, CD = codebooks.shape
    bB = 1024
    NB = 1

    down_w = (Wd1, bd1, Wd2, bd2, Wd3, bd3, Pin, Pin_b, codebooks)
    up_w = (Wu1, bu1, Wu2, bu2, Wu3, bu3)

    def full(a):
        return pl.BlockSpec(a.shape, lambda i: (0,) * a.ndim)

    M = pl.pallas_call(
        _decode_body,
        in_specs=[pl.BlockSpec(a.shape, lambda *_, _n=a.ndim: (0,) * _n)
                  for a in (codebooks, Pout, Pout_b)],
        out_specs=pl.BlockSpec((E, K, H), lambda *_: (0, 0, 0)),
        out_shape=jax.ShapeDtypeStruct((E, K, H), jnp.float32),
    )(codebooks, Pout, Pout_b)
    m_flat = M.reshape(E * K, H)

    Bc = B // NB
    u_parts, idx_parts = [], []
    for nb in range(NB):
        xc = lax.slice_in_dim(x, nb * Bc, (nb + 1) * Bc, axis=0)
        dT = pl.pallas_call(
            _down_body,
            grid=(Bc // bB,),
            in_specs=[pl.BlockSpec((bB, D), lambda i: (i, 0))] +
                     [full(w) for w in down_w],
            out_specs=pl.BlockSpec((E, bB // _SLAB, K, _SLAB),
                                   lambda i: (0, i, 0, 0)),
            out_shape=jax.ShapeDtypeStruct((E, Bc // _SLAB, K, _SLAB),
                                           jnp.float32),
            compiler_params=pltpu.CompilerParams(
                dimension_semantics=("arbitrary",)),
        )(xc, *down_w)

        idx_c, q_c = _sc_argmin_gather(dT, m_flat, E, K, H)

        u_c = pl.pallas_call(
            _up_body,
            grid=(Bc // bB,),
            in_specs=[pl.BlockSpec((E, bB, H), lambda i: (0, i, 0))] +
                     [full(w) for w in up_w],
            out_specs=pl.BlockSpec((bB, D), lambda i: (i, 0)),
            out_shape=jax.ShapeDtypeStruct((Bc, D), jnp.float32),
            compiler_params=pltpu.CompilerParams(
                dimension_semantics=("arbitrary",)),
        )(q_c, *up_w)
        u_parts.append(u_c)
        idx_parts.append(idx_c)

    u = u_parts[0] if NB == 1 else jnp.concatenate(u_parts, axis=0)
    idx = idx_parts[0] if NB == 1 else jnp.concatenate(idx_parts, axis=1)
    return u, idx.T, jnp.zeros((), jnp.float32)


# speedup vs baseline: 2.4514x; 1.0081x over previous
import functools

import jax
import jax.numpy as jnp
from jax import lax
from jax.experimental import pallas as pl
from jax.experimental.pallas import tpu as pltpu
from jax.experimental.pallas import tpu_sc as plsc

_NC, _NS, _L = 2, 16, 16
_SLAB = 128


def _down_body(x_ref, wd1, bd1, wd2, bd2, wd3, bd3, pin, pinb, cb, pout,
               poutb, dT_ref, m_ref):
    f32 = jnp.float32
    E_, K_, CD_ = cb.shape

    @pl.when(pl.program_id(0) == 0)
    def _():
        for i in range(E_):
            m_ref[i] = (jnp.dot(cb[i], pout[i], preferred_element_type=f32)
                        + poutb[i]) * (1.0 / E_)

    xb = x_ref[...]
    h = jnp.maximum(jnp.dot(xb, wd1[...], preferred_element_type=f32) + bd1[...], 0.0)
    h = jnp.maximum(jnp.dot(h, wd2[...], preferred_element_type=f32) + bd2[...], 0.0)
    h = jnp.dot(h, wd3[...], preferred_element_type=f32) + bd3[...]
    E, K, CD = cb.shape
    bB = h.shape[0]
    nslab = bB // _SLAB
    rows = []
    for i in range(E):
        z = jnp.dot(h, pin[i], preferred_element_type=f32) + pinb[i]
        cbi = cb[i]
        cb2 = jnp.sum(cbi * cbi, axis=1)[:, None]
        slabs = []
        for t in range(nslab):
            zt = z[t * _SLAB:(t + 1) * _SLAB]
            cross = lax.dot_general(cbi, zt, (((1,), (1,)), ((), ())),
                                    preferred_element_type=f32)
            slabs.append(cb2 - 2.0 * cross)
        rows.append(jnp.stack(slabs, axis=0))
    dT_ref[...] = jnp.stack(rows, axis=0)


def _up_body(q_ref, wu1, bu1, wu2, bu2, wu3, bu3, u_ref):
    f32 = jnp.float32
    E = q_ref.shape[0]
    hq = q_ref[0]
    for i in range(1, E):
        hq = hq + q_ref[i]
    u = jnp.maximum(jnp.dot(hq, wu1[...], preferred_element_type=f32) + bu1[...], 0.0)
    u = jnp.maximum(jnp.dot(u, wu2[...], preferred_element_type=f32) + bu2[...], 0.0)
    u = jnp.dot(u, wu3[...], preferred_element_type=f32) + bu3[...]
    u_ref[...] = jnp.clip(u, -1.0, 1.0)


def _sc_argmin_gather(dT, m_flat, E, K, H):
    B = dT.shape[1] * _SLAB
    NW = _NC * _NS
    per_w = (B // _SLAB) // NW
    nsub = _SLAB // _L
    mesh = plsc.VectorSubcoreMesh(core_axis_name="c", subcore_axis_name="s")

    n_items = E * per_w

    @functools.partial(
        pl.kernel, mesh=mesh,
        out_type=[jax.ShapeDtypeStruct((E, B), jnp.int32),
                  jax.ShapeDtypeStruct((E, B, H), jnp.float32)],
        scratch_types=[pltpu.VMEM((K, _SLAB), jnp.float32),
                       pltpu.VMEM((K, _SLAB), jnp.float32),
                       pltpu.VMEM((1, _SLAB), jnp.int32),
                       pltpu.VMEM((1, _SLAB), jnp.int32),
                       pltpu.VMEM((_SLAB,), jnp.int32),
                       pltpu.VMEM((_SLAB,), jnp.int32),
                       pltpu.VMEM((_SLAB, H), jnp.float32),
                       pltpu.VMEM((_SLAB, H), jnp.float32),
                       pltpu.VMEM_SHARED((1024, H), jnp.float32),
                       pltpu.SemaphoreType.DMA,
                       pltpu.SemaphoreType.DMA,
                       pltpu.SemaphoreType.DMA,
                       pltpu.SemaphoreType.DMA,
                       pltpu.SemaphoreType.DMA,
                       pltpu.SemaphoreType.DMA,
                       pltpu.SemaphoreType.DMA,
                       pltpu.SemaphoreType.DMA],
    )
    def k(dT_hbm, m_hbm, idx_hbm, q_hbm,
          d_v0, d_v1, i_v0, i_v1, g_v0, g_v1, r_v0, r_v1, m_sh,
          s_in0, s_in1, s_ix0, s_ix1, s_g0, s_g1, s_q0, s_q1):
        wid = lax.axis_index("s") * _NC + lax.axis_index("c")
        @pl.when(lax.axis_index("s") == 0)
        def _():
            pltpu.sync_copy(m_hbm, m_sh.at[pl.ds(0, E * K)])
        plsc.subcore_barrier()
        d_v = (d_v0, d_v1)
        i_v = (i_v0, i_v1)
        g_v = (g_v0, g_v1)
        r_v = (r_v0, r_v1)
        s_in = (s_in0, s_in1)
        s_ix = (s_ix0, s_ix1)
        s_g = (s_g0, s_g1)
        s_q = (s_q0, s_q1)

        def slab_src(item):
            e, j = divmod(item, per_w)
            base = (wid * per_w + j) * _SLAB
            return e, base

        def in_copy(item, p):
            e, base = slab_src(item)
            return pltpu.make_async_copy(
                dT_hbm.at[e, base // _SLAB], d_v[p], s_in[p])

        def ix_copy(item, p):
            e, base = slab_src(item)
            return pltpu.make_async_copy(
                i_v[p], idx_hbm.at[pl.ds(e, 1), pl.ds(base, _SLAB)], s_ix[p])

        def q_copy(item, p):
            e, base = slab_src(item)
            return pltpu.make_async_copy(
                r_v[p], q_hbm.at[e, pl.ds(base, _SLAB), :], s_q[p])

        UN = 4
        inf = jnp.full((_L,), jnp.inf, jnp.float32)
        zero = jnp.zeros((_L,), jnp.int32)

        in_copy(0, 0).start()
        for item in range(n_items):
            p = item % 2
            e, base = slab_src(item)
            if item + 1 < n_items:
                in_copy(item + 1, 1 - p).start()
            in_copy(item, p).wait()

            def body(kk, carry, _d=d_v[p]):
                outs = []
                for s in range(nsub):
                    best, besti = carry[s]
                    for u in range(UN):
                        krow = kk * UN + u
                        val = _d[krow, pl.ds(s * _L, _L)]
                        pred = val < best
                        best = jnp.where(pred, val, best)
                        besti = jnp.where(
                            pred, jnp.full((_L,), krow, jnp.int32), besti)
                    outs.append((best, besti))
                return tuple(outs)

            carry = lax.fori_loop(
                0, K // UN, body, tuple((inf, zero) for _ in range(nsub)))

            if item >= 2:
                ix_copy(item - 2, p).wait()
            for s in range(nsub):
                sl = pl.ds(s * _L, _L)
                i_v[p][0, sl] = carry[s][1]
                g_v[p][sl] = carry[s][1] + e * K
            ix_copy(item, p).start()

            if item >= 2:
                q_copy(item - 2, p).wait()
            pltpu.make_async_copy(m_sh.at[g_v[p]], r_v[p], s_g[p]).start()
            if item >= 1:
                pm = (item - 1) % 2
                pltpu.make_async_copy(
                    m_sh.at[g_v[pm]], r_v[pm], s_g[pm]).wait()
                q_copy(item - 1, pm).start()

        pl_last = (n_items - 1) % 2
        pltpu.make_async_copy(
            m_sh.at[g_v[pl_last]], r_v[pl_last], s_g[pl_last]).wait()
        q_copy(n_items - 1, pl_last).start()
        for item in (n_items - 2, n_items - 1):
            p = item % 2
            ix_copy(item, p).wait()
            q_copy(item, p).wait()

    return k(dT, m_flat)


def kernel(x, Wd1, bd1, Wd2, bd2, Wd3, bd3, Pin, Pin_b, codebooks, Pout,
           Pout_b, Wu1, bu1, Wu2, bu2, Wu3, bu3):
    B, D = x.shape
    H = Wd3.shape[1]
    E, K, CD = codebooks.shape
    bB = 1024
    NB = 1

    down_w = (Wd1, bd1, Wd2, bd2, Wd3, bd3, Pin, Pin_b, codebooks, Pout,
              Pout_b)
    up_w = (Wu1, bu1, Wu2, bu2, Wu3, bu3)

    def full(a):
        return pl.BlockSpec(a.shape, lambda i: (0,) * a.ndim)

    Bc = B // NB
    u_parts, idx_parts = [], []
    for nb in range(NB):
        xc = lax.slice_in_dim(x, nb * Bc, (nb + 1) * Bc, axis=0)
        dT, M = pl.pallas_call(
            _down_body,
            grid=(Bc // bB,),
            in_specs=[pl.BlockSpec((bB, D), lambda i: (i, 0))] +
                     [full(w) for w in down_w],
            out_specs=[pl.BlockSpec((E, bB // _SLAB, K, _SLAB),
                                    lambda i: (0, i, 0, 0)),
                       pl.BlockSpec((E, K, H), lambda i: (0, 0, 0))],
            out_shape=[jax.ShapeDtypeStruct((E, Bc // _SLAB, K, _SLAB),
                                            jnp.float32),
                       jax.ShapeDtypeStruct((E, K, H), jnp.float32)],
            compiler_params=pltpu.CompilerParams(
                dimension_semantics=("arbitrary",)),
        )(xc, *down_w)
        m_flat = M.reshape(E * K, H)

        idx_c, q_c = _sc_argmin_gather(dT, m_flat, E, K, H)

        u_c = pl.pallas_call(
            _up_body,
            grid=(Bc // bB,),
            in_specs=[pl.BlockSpec((E, bB, H), lambda i: (0, i, 0))] +
                     [full(w) for w in up_w],
            out_specs=pl.BlockSpec((bB, D), lambda i: (i, 0)),
            out_shape=jax.ShapeDtypeStruct((Bc, D), jnp.float32),
            compiler_params=pltpu.CompilerParams(
                dimension_semantics=("arbitrary",)),
        )(q_c, *up_w)
        u_parts.append(u_c)
        idx_parts.append(idx_c)

    u = u_parts[0] if NB == 1 else jnp.concatenate(u_parts, axis=0)
    idx = idx_parts[0] if NB == 1 else jnp.concatenate(idx_parts, axis=1)
    return u, idx.T, jnp.zeros((), jnp.float32)
